# baseline (speedup 1.0000x reference)
import functools

import jax
import jax.numpy as jnp
from jax import lax
from jax.experimental import pallas as pl
from jax.experimental.pallas import tpu as pltpu
from jax.experimental.pallas import tpu_sc as plsc

N = 10000
E = 320000
D = 128
ED = 16
SCW = 128
NCONV = 3
NCL = 2

NC = 2
NS = 16
NW = NC * NS
EPW = E // NW
CH = 80
NCH = EPW // CH
NPAD = 10240
RPT = NPAD // NS
NSEG = D // 16


def _leaky(h):
    return jnp.where(h >= 0, h, 0.01 * h)


_PL = tuple(32 * (c // 16) + (c % 16) for c in range(D // 2))
_PH = tuple(p + 16 for p in _PL)


_BE = 3200


def _rb16(x):
    return lax.shift_right_logical(
        x + 0x7FFF + (lax.shift_right_logical(x, 16) & 1), 16
    )


def _ea_body(ea2_ref, wl_ref, wh_ref, bl_ref, bh_ref, o_ref):
    a = (
        jnp.dot(ea2_ref[...], wl_ref[...], preferred_element_type=jnp.float32)
        + bl_ref[...]
    )
    bq = (
        jnp.dot(ea2_ref[...], wh_ref[...], preferred_element_type=jnp.float32)
        + bh_ref[...]
    )
    ai = lax.bitcast_convert_type(a, jnp.int32)
    bi = lax.bitcast_convert_type(bq, jnp.int32)
    o_ref[...] = _rb16(ai) | lax.shift_left(_rb16(bi), 16)


_BE2 = 1600


def _ea_layer(ea2, wl, wh, bl, bh):
    return pl.pallas_call(
        _ea_body,
        grid=(E // 2 // _BE2,),
        in_specs=[
            pl.BlockSpec((_BE2, 2 * ED), lambda e: (e, 0)),
            pl.BlockSpec((2 * ED, D), lambda e: (0, 0)),
            pl.BlockSpec((2 * ED, D), lambda e: (0, 0)),
            pl.BlockSpec((1, D), lambda e: (0, 0)),
            pl.BlockSpec((1, D), lambda e: (0, 0)),
        ],
        out_specs=pl.BlockSpec((_BE2, D), lambda e: (e, 0)),
        out_shape=jax.ShapeDtypeStruct((E // 2, D), jnp.int32),
    )(ea2, wl, wh, bl, bh)


_NQUAD = (NCH + 3) // 4 if NCH % 4 else NCH // 4


def _make_sc_msg():
    mesh = plsc.VectorSubcoreMesh(
        core_axis_name="c", subcore_axis_name="s", num_cores=NC, num_subcores=NS
    )

    @functools.partial(
        pl.kernel,
        mesh=mesh,
        compiler_params=pltpu.CompilerParams(needs_layout_passes=False),
        out_type=jax.ShapeDtypeStruct((NC, NPAD, D), jnp.float32),
        scratch_types=[
            pltpu.VMEM_SHARED((NPAD, D), jnp.float32),
            [pltpu.VMEM((CH,), jnp.int32)] * 2,
            [pltpu.VMEM((CH,), jnp.int32)] * 4,
            [pltpu.VMEM((CH // 2, D), jnp.int32)] * 2,
            [pltpu.VMEM((CH, D), jnp.float32)] * 2,
            [pltpu.SemaphoreType.DMA] * 2,
            [pltpu.SemaphoreType.DMA] * 4,
            [pltpu.SemaphoreType.DMA] * 2,
            [pltpu.SemaphoreType.DMA] * 2,
            [pltpu.SemaphoreType.DMA] * 2,
            pltpu.SemaphoreType.DMA,
        ],
    )
    def body(h_hbm, ea_hbm, src_hbm, dst_hbm, zeros_hbm, out_hbm,
             agg_sh, src_v, dst_v, ea_v, rows_v,
             sem_src, sem_dst, sem_ea, sem_g, sem_sc, sem_z):
        c = lax.axis_index("c")
        s = lax.axis_index("s")
        wid = s * NC + c
        base = wid * EPW
        base2 = wid * (EPW // 2)

        pltpu.async_copy(zeros_hbm, agg_sh.at[pl.ds(s * RPT, RPT)], sem_z)

        def load_src(ci, b):
            pltpu.async_copy(
                src_hbm.at[pl.ds(base + ci * CH, CH)], src_v[b], sem_src[b]
            )

        def load_dst(ci, d):
            pltpu.async_copy(
                dst_hbm.at[pl.ds(base + ci * CH, CH)], dst_v[d], sem_dst[d]
            )

        def wait_src(ci, b):
            pltpu.make_async_copy(
                src_hbm.at[pl.ds(base + ci * CH, CH)], src_v[b], sem_src[b]
            ).wait()

        def wait_dst(ci, d):
            pltpu.make_async_copy(
                dst_hbm.at[pl.ds(base + ci * CH, CH)], dst_v[d], sem_dst[d]
            ).wait()

        def start_gather_ea(ci, b):
            pltpu.async_copy(
                ea_hbm.at[pl.ds(base2 + ci * (CH // 2), CH // 2)], ea_v[b],
                sem_ea[b],
            )
            pltpu.async_copy(h_hbm.at[src_v[b]], rows_v[b], sem_g[b])

        def wait_gather_ea(ci, b):
            pltpu.make_async_copy(
                ea_hbm.at[pl.ds(base2 + ci * (CH // 2), CH // 2)], ea_v[b],
                sem_ea[b],
            ).wait()
            pltpu.make_async_copy(h_hbm.at[src_v[b]], rows_v[b], sem_g[b]).wait()

        def wait_scatter(b):
            pltpu.make_async_copy(
                rows_v[b], agg_sh.at[dst_v[0]], sem_sc[b]
            ).wait()

        load_src(0, 0)
        load_src(1, 1)
        load_dst(0, 0)
        load_dst(1, 1)
        wait_src(0, 0)
        start_gather_ea(0, 0)
        pltpu.make_async_copy(
            zeros_hbm, agg_sh.at[pl.ds(s * RPT, RPT)], sem_z
        ).wait()
        plsc.subcore_barrier()

        def slot(ci, j, first=False, do_next=True, do_next2=True):
            b = j % 2
            dj = j % 4
            if do_next:
                wait_src(ci + 1, 1 - b)
                if not first:
                    wait_scatter(1 - b)
                start_gather_ea(ci + 1, 1 - b)
            if do_next2:
                load_dst(ci + 2, (dj + 2) % 4)
            wait_gather_ea(ci, b)
            if do_next2:
                load_src(ci + 2, b)
            wait_dst(ci, dj)

            @plsc.parallel_loop(0, CH // 2, 1, unroll=2)
            def _(wr):
                for hh in range(2):
                    r = 2 * wr + hh
                    for g in range(D // 32):
                        w = ea_v[b][wr, pl.ds(hh * 64 + g * 16, 16)]
                        f_lo = plsc.bitcast(
                            lax.shift_left(w, 16), jnp.float32
                        )
                        f_hi = plsc.bitcast(
                            w & jnp.int32(-65536), jnp.float32
                        )
                        sl_lo = pl.ds(g * 32, 16)
                        sl_hi = pl.ds(g * 32 + 16, 16)
                        rows_v[b][r, sl_lo] = jnp.maximum(
                            rows_v[b][r, sl_lo] + f_lo, 0.0
                        )
                        rows_v[b][r, sl_hi] = jnp.maximum(
                            rows_v[b][r, sl_hi] + f_hi, 0.0
                        )

            pltpu.async_copy(
                rows_v[b], agg_sh.at[dst_v[dj]], sem_sc[b], add=True
            )

        slot(0, 0, first=True)
        for j in range(1, 4):
            slot(j, j)

        def quad(q, carry):
            for j in range(4):
                slot(4 * q + j, j)
            return carry

        lax.fori_loop(1, NCH // 4 - 1, quad, 0)

        for ci in range(4 * (NCH // 4 - 1), NCH):
            slot(ci, ci % 4, do_next=ci + 1 < NCH, do_next2=ci + 2 < NCH)

        for b in range(2):
            wait_scatter(b)

        plsc.subcore_barrier()
        pltpu.sync_copy(
            agg_sh.at[pl.ds(s * RPT, RPT)],
            out_hbm.at[c, pl.ds(s * RPT, RPT)],
        )

    return body


_SC_MSG = _make_sc_msg()


def _node_body(eps1_ref, h_ref, agg_ref, wn_ref, bn_ref, g_ref, b_ref, o_ref):
    h2 = eps1_ref[0] * h_ref[...] + (agg_ref[0, :N] + agg_ref[1, :N])
    t = jnp.dot(h2, wn_ref[...], preferred_element_type=jnp.float32) + bn_ref[...]
    mu = jnp.mean(t, axis=0, keepdims=True)
    var = jnp.mean((t - mu) ** 2, axis=0, keepdims=True)
    xn = (t - mu) * lax.rsqrt(var + 1e-5) * g_ref[...] + b_ref[...]
    o_ref[...] = _leaky(_leaky(xn))


def _node_update(eps1, h, agg, wn, bn, gamma, beta):
    return pl.pallas_call(
        _node_body,
        in_specs=[pl.BlockSpec(memory_space=pltpu.SMEM)]
        + [pl.BlockSpec()] * 6,
        out_specs=pl.BlockSpec(),
        out_shape=jax.ShapeDtypeStruct((N, D), jnp.float32),
    )(eps1, h, agg, wn, bn, gamma, beta)


def _node_cls_body(eps1_ref, h_ref, agg_ref, wn_ref, bn_ref, g_ref, b_ref,
                   wc1_ref, bc1_ref, wc_ref, bc_ref, wf_ref, bf_ref, o_ref):
    h2 = eps1_ref[0] * h_ref[...] + (agg_ref[0, :N] + agg_ref[1, :N])
    t = jnp.dot(h2, wn_ref[...], preferred_element_type=jnp.float32) + bn_ref[...]
    mu = jnp.mean(t, axis=0, keepdims=True)
    var = jnp.mean((t - mu) ** 2, axis=0, keepdims=True)
    xn = (t - mu) * lax.rsqrt(var + 1e-5) * g_ref[...] + b_ref[...]
    g = _leaky(_leaky(xn))
    g = jnp.dot(g, wc1_ref[...], preferred_element_type=jnp.float32)
    g = g + bc1_ref[...]
    for i in range(NCL):
        g = jnp.dot(g, wc_ref[i], preferred_element_type=jnp.float32) + bc_ref[i]
        g = _leaky(g)
    z = jnp.dot(g, wf_ref[...], preferred_element_type=jnp.float32) + bf_ref[...]
    o_ref[...] = jax.nn.sigmoid(z)


def _node_cls(eps1, h, agg, wn, bn, gamma, beta, Wc1, bc1, Wc, bc, Wf, bf):
    return pl.pallas_call(
        _node_cls_body,
        in_specs=[pl.BlockSpec(memory_space=pltpu.SMEM)]
        + [pl.BlockSpec()] * 12,
        out_specs=pl.BlockSpec(),
        out_shape=jax.ShapeDtypeStruct((N, 1), jnp.float32),
    )(eps1, h, agg, wn, bn, gamma, beta, Wc1, bc1, Wc, bc, Wf, bf)


def kernel(x, edge_index, edge_attr, batch, We, be, eps, Wn, bn, gamma, beta,
           Wc1, bc1, Wc, bc, Wf, bf):
    src = edge_index[0]
    dst = edge_index[1]
    zeros = jnp.zeros((RPT, D), jnp.float32)
    pl_idx = jnp.array(_PL, dtype=jnp.int32)
    ph_idx = jnp.array(_PH, dtype=jnp.int32)
    ea2 = edge_attr.reshape(E // 2, 2 * ED)
    z = jnp.zeros((ED, D // 2), jnp.float32)
    eas = []
    for i in range(NCONV):
        wpl = We[i][:, pl_idx]
        wph = We[i][:, ph_idx]
        wl = jnp.block([[wpl, z], [z, wpl]])
        wh = jnp.block([[wph, z], [z, wph]])
        bl = jnp.concatenate([be[i][pl_idx], be[i][pl_idx]]).reshape(1, D)
        bh = jnp.concatenate([be[i][ph_idx], be[i][ph_idx]]).reshape(1, D)
        eas.append(_ea_layer(ea2, wl, wh, bl, bh))
    h = x
    for i in range(NCONV - 1):
        agg = _SC_MSG(h, eas[i], src, dst, zeros)
        h = _node_update(
            (1.0 + eps[i]).reshape(1),
            h, agg, Wn[i],
            bn[i].reshape(1, D),
            gamma[i].reshape(1, D),
            beta[i].reshape(1, D),
        )
    i = NCONV - 1
    agg = _SC_MSG(h, eas[i], src, dst, zeros)
    return _node_cls(
        (1.0 + eps[i]).reshape(1),
        h, agg, Wn[i],
        bn[i].reshape(1, D),
        gamma[i].reshape(1, D),
        beta[i].reshape(1, D),
        Wc1, bc1.reshape(1, SCW), Wc, bc, Wf, bf.reshape(1, 1),
    )


# speedup vs baseline: 1.0114x; 1.0114x over previous
import functools

import jax
import jax.numpy as jnp
from jax import lax
from jax.experimental import pallas as pl
from jax.experimental.pallas import tpu as pltpu
from jax.experimental.pallas import tpu_sc as plsc

N = 10000
E = 320000
D = 128
ED = 16
SCW = 128
NCONV = 3
NCL = 2

NC = 2
NS = 16
NW = NC * NS
EPW = E // NW
CH = 80
NCH = EPW // CH
NPAD = 10240
RPT = NPAD // NS
NSEG = D // 16


def _leaky(h):
    return jnp.where(h >= 0, h, 0.01 * h)


_PL = tuple(32 * (c // 16) + (c % 16) for c in range(D // 2))
_PH = tuple(p + 16 for p in _PL)


_BE = 3200


def _rb16(x):
    return lax.shift_right_logical(
        x + 0x7FFF + (lax.shift_right_logical(x, 16) & 1), 16
    )


def _ea_body(ea2_ref, wl_ref, wh_ref, bl_ref, bh_ref, o_ref):
    a = (
        jnp.dot(ea2_ref[...], wl_ref[...], preferred_element_type=jnp.float32)
        + bl_ref[...]
    )
    bq = (
        jnp.dot(ea2_ref[...], wh_ref[...], preferred_element_type=jnp.float32)
        + bh_ref[...]
    )
    ai = lax.bitcast_convert_type(a, jnp.int32)
    bi = lax.bitcast_convert_type(bq, jnp.int32)
    o_ref[...] = _rb16(ai) | lax.shift_left(_rb16(bi), 16)


_BE2 = 1600


def _ea_layer(ea2, wl, wh, bl, bh):
    return pl.pallas_call(
        _ea_body,
        grid=(E // 2 // _BE2,),
        in_specs=[
            pl.BlockSpec((_BE2, 2 * ED), lambda e: (e, 0)),
            pl.BlockSpec((2 * ED, D), lambda e: (0, 0)),
            pl.BlockSpec((2 * ED, D), lambda e: (0, 0)),
            pl.BlockSpec((1, D), lambda e: (0, 0)),
            pl.BlockSpec((1, D), lambda e: (0, 0)),
        ],
        out_specs=pl.BlockSpec((_BE2, D), lambda e: (e, 0)),
        out_shape=jax.ShapeDtypeStruct((E // 2, D), jnp.int32),
    )(ea2, wl, wh, bl, bh)


_NQUAD = (NCH + 3) // 4 if NCH % 4 else NCH // 4


def _make_sc_msg():
    mesh = plsc.VectorSubcoreMesh(
        core_axis_name="c", subcore_axis_name="s", num_cores=NC, num_subcores=NS
    )

    @functools.partial(
        pl.kernel,
        mesh=mesh,
        compiler_params=pltpu.CompilerParams(needs_layout_passes=False),
        out_type=jax.ShapeDtypeStruct((NC, NPAD, D), jnp.float32),
        scratch_types=[
            pltpu.VMEM_SHARED((NPAD, D), jnp.float32),
            [pltpu.VMEM((CH,), jnp.int32)] * 2,
            [pltpu.VMEM((CH,), jnp.int32)] * 4,
            [pltpu.VMEM((CH // 2, D), jnp.int32)] * 2,
            [pltpu.VMEM((CH, D), jnp.float32)] * 2,
            [pltpu.SemaphoreType.DMA] * 2,
            [pltpu.SemaphoreType.DMA] * 4,
            [pltpu.SemaphoreType.DMA] * 2,
            [pltpu.SemaphoreType.DMA] * 2,
            [pltpu.SemaphoreType.DMA] * 2,
            pltpu.SemaphoreType.DMA,
        ],
    )
    def body(h_hbm, ea_hbm, src_hbm, dst_hbm, zeros_hbm, out_hbm,
             agg_sh, src_v, dst_v, ea_v, rows_v,
             sem_src, sem_dst, sem_ea, sem_g, sem_sc, sem_z):
        c = lax.axis_index("c")
        s = lax.axis_index("s")
        wid = s * NC + c
        base = wid * EPW
        base2 = wid * (EPW // 2)

        pltpu.async_copy(zeros_hbm, agg_sh.at[pl.ds(s * RPT, RPT)], sem_z)

        def load_src(ci, b):
            pltpu.async_copy(
                src_hbm.at[pl.ds(base + ci * CH, CH)], src_v[b], sem_src[b]
            )

        def load_dst(ci, d):
            pltpu.async_copy(
                dst_hbm.at[pl.ds(base + ci * CH, CH)], dst_v[d], sem_dst[d]
            )

        def wait_src(ci, b):
            pltpu.make_async_copy(
                src_hbm.at[pl.ds(base + ci * CH, CH)], src_v[b], sem_src[b]
            ).wait()

        def wait_dst(ci, d):
            pltpu.make_async_copy(
                dst_hbm.at[pl.ds(base + ci * CH, CH)], dst_v[d], sem_dst[d]
            ).wait()

        def start_gather_ea(ci, b):
            pltpu.async_copy(
                ea_hbm.at[pl.ds(base2 + ci * (CH // 2), CH // 2)], ea_v[b],
                sem_ea[b],
            )
            pltpu.async_copy(h_hbm.at[src_v[b]], rows_v[b], sem_g[b])

        def wait_gather_ea(ci, b):
            pltpu.make_async_copy(
                ea_hbm.at[pl.ds(base2 + ci * (CH // 2), CH // 2)], ea_v[b],
                sem_ea[b],
            ).wait()
            pltpu.make_async_copy(h_hbm.at[src_v[b]], rows_v[b], sem_g[b]).wait()

        def wait_scatter(b):
            pltpu.make_async_copy(
                rows_v[b], agg_sh.at[dst_v[0]], sem_sc[b]
            ).wait()

        load_src(0, 0)
        load_src(1, 1)
        load_dst(0, 0)
        load_dst(1, 1)
        wait_src(0, 0)
        start_gather_ea(0, 0)
        pltpu.make_async_copy(
            zeros_hbm, agg_sh.at[pl.ds(s * RPT, RPT)], sem_z
        ).wait()
        plsc.subcore_barrier()

        def slot(ci, j):
            b = j % 2
            dj = j % 4

            @pl.when(ci + 1 < NCH)
            def _():
                wait_src(ci + 1, 1 - b)
                @pl.when(ci >= 1)
                def _():
                    wait_scatter(1 - b)

                start_gather_ea(ci + 1, 1 - b)

            @pl.when(ci + 2 < NCH)
            def _():
                load_dst(ci + 2, (dj + 2) % 4)

            wait_gather_ea(ci, b)

            @pl.when(ci + 2 < NCH)
            def _():
                load_src(ci + 2, b)

            wait_dst(ci, dj)

            @plsc.parallel_loop(0, CH // 2, 1, unroll=2)
            def _(wr):
                for hh in range(2):
                    r = 2 * wr + hh
                    for g in range(D // 32):
                        w = ea_v[b][wr, pl.ds(hh * 64 + g * 16, 16)]
                        f_lo = plsc.bitcast(
                            lax.shift_left(w, 16), jnp.float32
                        )
                        f_hi = plsc.bitcast(
                            w & jnp.int32(-65536), jnp.float32
                        )
                        sl_lo = pl.ds(g * 32, 16)
                        sl_hi = pl.ds(g * 32 + 16, 16)
                        rows_v[b][r, sl_lo] = jnp.maximum(
                            rows_v[b][r, sl_lo] + f_lo, 0.0
                        )
                        rows_v[b][r, sl_hi] = jnp.maximum(
                            rows_v[b][r, sl_hi] + f_hi, 0.0
                        )

            pltpu.async_copy(
                rows_v[b], agg_sh.at[dst_v[dj]], sem_sc[b], add=True
            )

        def quad(q, carry):
            for j in range(4):
                ci = 4 * q + j

                @pl.when(ci < NCH)
                def _():
                    slot(ci, j)
            return carry

        lax.fori_loop(0, _NQUAD, quad, 0)

        for b in range(2):
            wait_scatter(b)

        plsc.subcore_barrier()
        pltpu.sync_copy(
            agg_sh.at[pl.ds(s * RPT, RPT)],
            out_hbm.at[c, pl.ds(s * RPT, RPT)],
        )

    return body


_SC_MSG = _make_sc_msg()


def _node_body(eps1_ref, h_ref, agg_ref, wn_ref, bn_ref, g_ref, b_ref, o_ref):
    h2 = eps1_ref[0] * h_ref[...] + (agg_ref[0, :N] + agg_ref[1, :N])
    t = jnp.dot(h2, wn_ref[...], preferred_element_type=jnp.float32) + bn_ref[...]
    mu = jnp.mean(t, axis=0, keepdims=True)
    var = jnp.mean((t - mu) ** 2, axis=0, keepdims=True)
    xn = (t - mu) * lax.rsqrt(var + 1e-5) * g_ref[...] + b_ref[...]
    o_ref[...] = _leaky(_leaky(xn))


def _node_update(eps1, h, agg, wn, bn, gamma, beta):
    return pl.pallas_call(
        _node_body,
        in_specs=[pl.BlockSpec(memory_space=pltpu.SMEM)]
        + [pl.BlockSpec()] * 6,
        out_specs=pl.BlockSpec(),
        out_shape=jax.ShapeDtypeStruct((N, D), jnp.float32),
    )(eps1, h, agg, wn, bn, gamma, beta)


def _node_cls_body(eps1_ref, h_ref, agg_ref, wn_ref, bn_ref, g_ref, b_ref,
                   wc1_ref, bc1_ref, wc_ref, bc_ref, wf_ref, bf_ref, o_ref):
    h2 = eps1_ref[0] * h_ref[...] + (agg_ref[0, :N] + agg_ref[1, :N])
    t = jnp.dot(h2, wn_ref[...], preferred_element_type=jnp.float32) + bn_ref[...]
    mu = jnp.mean(t, axis=0, keepdims=True)
    var = jnp.mean((t - mu) ** 2, axis=0, keepdims=True)
    xn = (t - mu) * lax.rsqrt(var + 1e-5) * g_ref[...] + b_ref[...]
    g = _leaky(_leaky(xn))
    g = jnp.dot(g, wc1_ref[...], preferred_element_type=jnp.float32)
    g = g + bc1_ref[...]
    for i in range(NCL):
        g = jnp.dot(g, wc_ref[i], preferred_element_type=jnp.float32) + bc_ref[i]
        g = _leaky(g)
    z = jnp.dot(g, wf_ref[...], preferred_element_type=jnp.float32) + bf_ref[...]
    o_ref[...] = jax.nn.sigmoid(z)


def _node_cls(eps1, h, agg, wn, bn, gamma, beta, Wc1, bc1, Wc, bc, Wf, bf):
    return pl.pallas_call(
        _node_cls_body,
        in_specs=[pl.BlockSpec(memory_space=pltpu.SMEM)]
        + [pl.BlockSpec()] * 12,
        out_specs=pl.BlockSpec(),
        out_shape=jax.ShapeDtypeStruct((N, 1), jnp.float32),
    )(eps1, h, agg, wn, bn, gamma, beta, Wc1, bc1, Wc, bc, Wf, bf)


def kernel(x, edge_index, edge_attr, batch, We, be, eps, Wn, bn, gamma, beta,
           Wc1, bc1, Wc, bc, Wf, bf):
    src = edge_index[0]
    dst = edge_index[1]
    zeros = jnp.zeros((RPT, D), jnp.float32)
    pl_idx = jnp.array(_PL, dtype=jnp.int32)
    ph_idx = jnp.array(_PH, dtype=jnp.int32)
    ea2 = edge_attr.reshape(E // 2, 2 * ED)
    z = jnp.zeros((ED, D // 2), jnp.float32)
    eas = []
    for i in range(NCONV):
        wpl = We[i][:, pl_idx]
        wph = We[i][:, ph_idx]
        wl = jnp.block([[wpl, z], [z, wpl]])
        wh = jnp.block([[wph, z], [z, wph]])
        bl = jnp.concatenate([be[i][pl_idx], be[i][pl_idx]]).reshape(1, D)
        bh = jnp.concatenate([be[i][ph_idx], be[i][ph_idx]]).reshape(1, D)
        eas.append(_ea_layer(ea2, wl, wh, bl, bh))
    h = x
    for i in range(NCONV - 1):
        agg = _SC_MSG(h, eas[i], src, dst, zeros)
        h = _node_update(
            (1.0 + eps[i]).reshape(1),
            h, agg, Wn[i],
            bn[i].reshape(1, D),
            gamma[i].reshape(1, D),
            beta[i].reshape(1, D),
        )
    i = NCONV - 1
    agg = _SC_MSG(h, eas[i], src, dst, zeros)
    return _node_cls(
        (1.0 + eps[i]).reshape(1),
        h, agg, Wn[i],
        bn[i].reshape(1, D),
        gamma[i].reshape(1, D),
        beta[i].reshape(1, D),
        Wc1, bc1.reshape(1, SCW), Wc, bc, Wf, bf.reshape(1, 1),
    )


# speedup vs baseline: 1.0171x; 1.0056x over previous
import functools

import jax
import jax.numpy as jnp
from jax import lax
from jax.experimental import pallas as pl
from jax.experimental.pallas import tpu as pltpu
from jax.experimental.pallas import tpu_sc as plsc

N = 10000
E = 320000
D = 128
ED = 16
SCW = 128
NCONV = 3
NCL = 2

NC = 2
NS = 16
NW = NC * NS
EPW = E // NW
CH = 80
NCH = EPW // CH
NPAD = 10240
RPT = NPAD // NS
NSEG = D // 16


def _leaky(h):
    return jnp.where(h >= 0, h, 0.01 * h)


_PL = tuple(32 * (c // 16) + (c % 16) for c in range(D // 2))
_PH = tuple(p + 16 for p in _PL)


_BE = 3200


def _rb16(x):
    return lax.shift_right_logical(
        x + 0x7FFF + (lax.shift_right_logical(x, 16) & 1), 16
    )


def _ea_body(ea2_ref, wl_ref, wh_ref, bl_ref, bh_ref, o_ref):
    a = (
        jnp.dot(ea2_ref[...], wl_ref[...], preferred_element_type=jnp.float32)
        + bl_ref[...]
    )
    bq = (
        jnp.dot(ea2_ref[...], wh_ref[...], preferred_element_type=jnp.float32)
        + bh_ref[...]
    )
    ai = lax.bitcast_convert_type(a, jnp.int32)
    bi = lax.bitcast_convert_type(bq, jnp.int32)
    o_ref[...] = _rb16(ai) | lax.shift_left(_rb16(bi), 16)


_BE2 = 1600


def _ea_layer(ea2, wl, wh, bl, bh):
    return pl.pallas_call(
        _ea_body,
        grid=(E // 2 // _BE2,),
        in_specs=[
            pl.BlockSpec((_BE2, 2 * ED), lambda e: (e, 0)),
            pl.BlockSpec((2 * ED, D), lambda e: (0, 0)),
            pl.BlockSpec((2 * ED, D), lambda e: (0, 0)),
            pl.BlockSpec((1, D), lambda e: (0, 0)),
            pl.BlockSpec((1, D), lambda e: (0, 0)),
        ],
        out_specs=pl.BlockSpec((_BE2, D), lambda e: (e, 0)),
        out_shape=jax.ShapeDtypeStruct((E // 2, D), jnp.int32),
    )(ea2, wl, wh, bl, bh)


_NQUAD = (NCH + 3) // 4 if NCH % 4 else NCH // 4


def _make_sc_msg():
    mesh = plsc.VectorSubcoreMesh(
        core_axis_name="c", subcore_axis_name="s", num_cores=NC, num_subcores=NS
    )

    @functools.partial(
        pl.kernel,
        mesh=mesh,
        compiler_params=pltpu.CompilerParams(needs_layout_passes=False),
        out_type=jax.ShapeDtypeStruct((NC, NPAD, D), jnp.float32),
        scratch_types=[
            pltpu.VMEM_SHARED((NPAD, D), jnp.float32),
            [pltpu.VMEM((CH,), jnp.int32)] * 2,
            [pltpu.VMEM((CH,), jnp.int32)] * 4,
            [pltpu.VMEM((CH // 2, D), jnp.int32)] * 2,
            [pltpu.VMEM((CH, D), jnp.float32)] * 2,
            [pltpu.SemaphoreType.DMA] * 2,
            [pltpu.SemaphoreType.DMA] * 4,
            [pltpu.SemaphoreType.DMA] * 2,
            [pltpu.SemaphoreType.DMA] * 2,
            [pltpu.SemaphoreType.DMA] * 2,
            pltpu.SemaphoreType.DMA,
        ],
    )
    def body(h_hbm, ea_hbm, src_hbm, dst_hbm, zeros_hbm, out_hbm,
             agg_sh, src_v, dst_v, ea_v, rows_v,
             sem_src, sem_dst, sem_ea, sem_g, sem_sc, sem_z):
        c = lax.axis_index("c")
        s = lax.axis_index("s")
        wid = s * NC + c
        base = wid * EPW
        base2 = wid * (EPW // 2)

        pltpu.async_copy(zeros_hbm, agg_sh.at[pl.ds(s * RPT, RPT)], sem_z)

        def load_src(ci, b):
            pltpu.async_copy(
                src_hbm.at[pl.ds(base + ci * CH, CH)], src_v[b], sem_src[b]
            )

        def load_dst(ci, d):
            pltpu.async_copy(
                dst_hbm.at[pl.ds(base + ci * CH, CH)], dst_v[d], sem_dst[d]
            )

        def wait_src(ci, b):
            pltpu.make_async_copy(
                src_hbm.at[pl.ds(base + ci * CH, CH)], src_v[b], sem_src[b]
            ).wait()

        def wait_dst(ci, d):
            pltpu.make_async_copy(
                dst_hbm.at[pl.ds(base + ci * CH, CH)], dst_v[d], sem_dst[d]
            ).wait()

        def start_gather_ea(ci, b):
            pltpu.async_copy(
                ea_hbm.at[pl.ds(base2 + ci * (CH // 2), CH // 2)], ea_v[b],
                sem_ea[b],
            )
            pltpu.async_copy(h_hbm.at[src_v[b]], rows_v[b], sem_g[b])

        def wait_gather_ea(ci, b):
            pltpu.make_async_copy(
                ea_hbm.at[pl.ds(base2 + ci * (CH // 2), CH // 2)], ea_v[b],
                sem_ea[b],
            ).wait()
            pltpu.make_async_copy(h_hbm.at[src_v[b]], rows_v[b], sem_g[b]).wait()

        def wait_scatter(b):
            pltpu.make_async_copy(
                rows_v[b], agg_sh.at[dst_v[0]], sem_sc[b]
            ).wait()

        load_src(0, 0)
        load_src(1, 1)
        load_dst(0, 0)
        load_dst(1, 1)
        wait_src(0, 0)
        start_gather_ea(0, 0)
        pltpu.make_async_copy(
            zeros_hbm, agg_sh.at[pl.ds(s * RPT, RPT)], sem_z
        ).wait()
        plsc.subcore_barrier()

        def slot(ci, j):
            b = j % 2
            dj = j % 4

            @pl.when(ci + 1 < NCH)
            def _():
                wait_src(ci + 1, 1 - b)
                @pl.when(ci >= 1)
                def _():
                    wait_scatter(1 - b)

                start_gather_ea(ci + 1, 1 - b)

            @pl.when(ci + 2 < NCH)
            def _():
                load_dst(ci + 2, (dj + 2) % 4)

            wait_gather_ea(ci, b)

            @pl.when(ci + 2 < NCH)
            def _():
                load_src(ci + 2, b)

            wait_dst(ci, dj)

            @plsc.parallel_loop(0, CH // 2, 1, unroll=4)
            def _(wr):
                for hh in range(2):
                    r = 2 * wr + hh
                    for g in range(D // 32):
                        w = ea_v[b][wr, pl.ds(hh * 64 + g * 16, 16)]
                        f_lo = plsc.bitcast(
                            lax.shift_left(w, 16), jnp.float32
                        )
                        f_hi = plsc.bitcast(
                            w & jnp.int32(-65536), jnp.float32
                        )
                        sl_lo = pl.ds(g * 32, 16)
                        sl_hi = pl.ds(g * 32 + 16, 16)
                        rows_v[b][r, sl_lo] = jnp.maximum(
                            rows_v[b][r, sl_lo] + f_lo, 0.0
                        )
                        rows_v[b][r, sl_hi] = jnp.maximum(
                            rows_v[b][r, sl_hi] + f_hi, 0.0
                        )

            pltpu.async_copy(
                rows_v[b], agg_sh.at[dst_v[dj]], sem_sc[b], add=True
            )

        def quad(q, carry):
            for j in range(4):
                ci = 4 * q + j

                @pl.when(ci < NCH)
                def _():
                    slot(ci, j)
            return carry

        lax.fori_loop(0, _NQUAD, quad, 0)

        for b in range(2):
            wait_scatter(b)

        plsc.subcore_barrier()
        pltpu.sync_copy(
            agg_sh.at[pl.ds(s * RPT, RPT)],
            out_hbm.at[c, pl.ds(s * RPT, RPT)],
        )

    return body


_SC_MSG = _make_sc_msg()


def _node_body(eps1_ref, h_ref, agg_ref, wn_ref, bn_ref, g_ref, b_ref, o_ref):
    h2 = eps1_ref[0] * h_ref[...] + (agg_ref[0, :N] + agg_ref[1, :N])
    t = jnp.dot(h2, wn_ref[...], preferred_element_type=jnp.float32) + bn_ref[...]
    mu = jnp.mean(t, axis=0, keepdims=True)
    var = jnp.mean((t - mu) ** 2, axis=0, keepdims=True)
    xn = (t - mu) * lax.rsqrt(var + 1e-5) * g_ref[...] + b_ref[...]
    o_ref[...] = _leaky(_leaky(xn))


def _node_update(eps1, h, agg, wn, bn, gamma, beta):
    return pl.pallas_call(
        _node_body,
        in_specs=[pl.BlockSpec(memory_space=pltpu.SMEM)]
        + [pl.BlockSpec()] * 6,
        out_specs=pl.BlockSpec(),
        out_shape=jax.ShapeDtypeStruct((N, D), jnp.float32),
    )(eps1, h, agg, wn, bn, gamma, beta)


def _node_cls_body(eps1_ref, h_ref, agg_ref, wn_ref, bn_ref, g_ref, b_ref,
                   wc1_ref, bc1_ref, wc_ref, bc_ref, wf_ref, bf_ref, o_ref):
    h2 = eps1_ref[0] * h_ref[...] + (agg_ref[0, :N] + agg_ref[1, :N])
    t = jnp.dot(h2, wn_ref[...], preferred_element_type=jnp.float32) + bn_ref[...]
    mu = jnp.mean(t, axis=0, keepdims=True)
    var = jnp.mean((t - mu) ** 2, axis=0, keepdims=True)
    xn = (t - mu) * lax.rsqrt(var + 1e-5) * g_ref[...] + b_ref[...]
    g = _leaky(_leaky(xn))
    g = jnp.dot(g, wc1_ref[...], preferred_element_type=jnp.float32)
    g = g + bc1_ref[...]
    for i in range(NCL):
        g = jnp.dot(g, wc_ref[i], preferred_element_type=jnp.float32) + bc_ref[i]
        g = _leaky(g)
    z = jnp.dot(g, wf_ref[...], preferred_element_type=jnp.float32) + bf_ref[...]
    o_ref[...] = jax.nn.sigmoid(z)


def _node_cls(eps1, h, agg, wn, bn, gamma, beta, Wc1, bc1, Wc, bc, Wf, bf):
    return pl.pallas_call(
        _node_cls_body,
        in_specs=[pl.BlockSpec(memory_space=pltpu.SMEM)]
        + [pl.BlockSpec()] * 12,
        out_specs=pl.BlockSpec(),
        out_shape=jax.ShapeDtypeStruct((N, 1), jnp.float32),
    )(eps1, h, agg, wn, bn, gamma, beta, Wc1, bc1, Wc, bc, Wf, bf)


def kernel(x, edge_index, edge_attr, batch, We, be, eps, Wn, bn, gamma, beta,
           Wc1, bc1, Wc, bc, Wf, bf):
    src = edge_index[0]
    dst = edge_index[1]
    zeros = jnp.zeros((RPT, D), jnp.float32)
    pl_idx = jnp.array(_PL, dtype=jnp.int32)
    ph_idx = jnp.array(_PH, dtype=jnp.int32)
    ea2 = edge_attr.reshape(E // 2, 2 * ED)
    z = jnp.zeros((ED, D // 2), jnp.float32)
    eas = []
    for i in range(NCONV):
        wpl = We[i][:, pl_idx]
        wph = We[i][:, ph_idx]
        wl = jnp.block([[wpl, z], [z, wpl]])
        wh = jnp.block([[wph, z], [z, wph]])
        bl = jnp.concatenate([be[i][pl_idx], be[i][pl_idx]]).reshape(1, D)
        bh = jnp.concatenate([be[i][ph_idx], be[i][ph_idx]]).reshape(1, D)
        eas.append(_ea_layer(ea2, wl, wh, bl, bh))
    h = x
    for i in range(NCONV - 1):
        agg = _SC_MSG(h, eas[i], src, dst, zeros)
        h = _node_update(
            (1.0 + eps[i]).reshape(1),
            h, agg, Wn[i],
            bn[i].reshape(1, D),
            gamma[i].reshape(1, D),
            beta[i].reshape(1, D),
        )
    i = NCONV - 1
    agg = _SC_MSG(h, eas[i], src, dst, zeros)
    return _node_cls(
        (1.0 + eps[i]).reshape(1),
        h, agg, Wn[i],
        bn[i].reshape(1, D),
        gamma[i].reshape(1, D),
        beta[i].reshape(1, D),
        Wc1, bc1.reshape(1, SCW), Wc, bc, Wf, bf.reshape(1, 1),
    )


# speedup vs baseline: 1.0448x; 1.0273x over previous
import functools

import jax
import jax.numpy as jnp
from jax import lax
from jax.experimental import pallas as pl
from jax.experimental.pallas import tpu as pltpu
from jax.experimental.pallas import tpu_sc as plsc

N = 10000
E = 320000
D = 128
ED = 16
SCW = 128
NCONV = 3
NCL = 2

NC = 2
NS = 16
NW = NC * NS
EPW = E // NW
CH = 80
NCH = EPW // CH
NPAD = 10240
RPT = NPAD // NS
NSEG = D // 16


def _leaky(h):
    return jnp.where(h >= 0, h, 0.01 * h)


_PL = tuple(32 * (c // 16) + (c % 16) for c in range(D // 2))
_PH = tuple(p + 16 for p in _PL)


_BE = 3200


def _rb16(x):
    return lax.shift_right_logical(
        x + 0x7FFF + (lax.shift_right_logical(x, 16) & 1), 16
    )


def _ea_body(ea2_ref, wl_ref, wh_ref, bl_ref, bh_ref, o_ref):
    a = (
        jnp.dot(ea2_ref[...], wl_ref[...], preferred_element_type=jnp.float32)
        + bl_ref[...]
    )
    bq = (
        jnp.dot(ea2_ref[...], wh_ref[...], preferred_element_type=jnp.float32)
        + bh_ref[...]
    )
    ai = lax.bitcast_convert_type(a, jnp.int32)
    bi = lax.bitcast_convert_type(bq, jnp.int32)
    o_ref[...] = _rb16(ai) | lax.shift_left(_rb16(bi), 16)


_BE2 = 1600


def _ea_layer(ea2, wl, wh, bl, bh):
    return pl.pallas_call(
        _ea_body,
        grid=(E // 2 // _BE2,),
        in_specs=[
            pl.BlockSpec((_BE2, 2 * ED), lambda e: (e, 0)),
            pl.BlockSpec((2 * ED, D), lambda e: (0, 0)),
            pl.BlockSpec((2 * ED, D), lambda e: (0, 0)),
            pl.BlockSpec((1, D), lambda e: (0, 0)),
            pl.BlockSpec((1, D), lambda e: (0, 0)),
        ],
        out_specs=pl.BlockSpec((_BE2, D), lambda e: (e, 0)),
        out_shape=jax.ShapeDtypeStruct((E // 2, D), jnp.int32),
    )(ea2, wl, wh, bl, bh)


_N6 = (NCH + 5) // 6


def _make_sc_msg():
    mesh = plsc.VectorSubcoreMesh(
        core_axis_name="c", subcore_axis_name="s", num_cores=NC, num_subcores=NS
    )

    @functools.partial(
        pl.kernel,
        mesh=mesh,
        compiler_params=pltpu.CompilerParams(needs_layout_passes=False),
        out_type=jax.ShapeDtypeStruct((NC, NPAD, D), jnp.float32),
        scratch_types=[
            pltpu.VMEM_SHARED((NPAD, D), jnp.float32),
            [pltpu.VMEM((CH,), jnp.int32)] * 2,
            [pltpu.VMEM((CH,), jnp.int32)] * 6,
            [pltpu.VMEM((CH // 2, D), jnp.int32)] * 2,
            [pltpu.VMEM((CH, D), jnp.float32)] * 3,
            [pltpu.SemaphoreType.DMA] * 2,
            [pltpu.SemaphoreType.DMA] * 6,
            [pltpu.SemaphoreType.DMA] * 2,
            [pltpu.SemaphoreType.DMA] * 3,
            [pltpu.SemaphoreType.DMA] * 3,
            pltpu.SemaphoreType.DMA,
        ],
    )
    def body(h_hbm, ea_hbm, src_hbm, dst_hbm, zeros_hbm, out_hbm,
             agg_sh, src_v, dst_v, ea_v, rows_v,
             sem_src, sem_dst, sem_ea, sem_g, sem_sc, sem_z):
        c = lax.axis_index("c")
        s = lax.axis_index("s")
        wid = s * NC + c
        base = wid * EPW
        base2 = wid * (EPW // 2)

        pltpu.async_copy(zeros_hbm, agg_sh.at[pl.ds(s * RPT, RPT)], sem_z)

        def load_src(ci, b):
            pltpu.async_copy(
                src_hbm.at[pl.ds(base + ci * CH, CH)], src_v[b], sem_src[b]
            )

        def load_dst(ci, d):
            pltpu.async_copy(
                dst_hbm.at[pl.ds(base + ci * CH, CH)], dst_v[d], sem_dst[d]
            )

        def wait_src(ci, b):
            pltpu.make_async_copy(
                src_hbm.at[pl.ds(base + ci * CH, CH)], src_v[b], sem_src[b]
            ).wait()

        def wait_dst(ci, d):
            pltpu.make_async_copy(
                dst_hbm.at[pl.ds(base + ci * CH, CH)], dst_v[d], sem_dst[d]
            ).wait()

        def start_gather_ea(ci, b, r3):
            pltpu.async_copy(
                ea_hbm.at[pl.ds(base2 + ci * (CH // 2), CH // 2)], ea_v[b],
                sem_ea[b],
            )
            pltpu.async_copy(h_hbm.at[src_v[b]], rows_v[r3], sem_g[r3])

        def wait_gather_ea(ci, b, r3):
            pltpu.make_async_copy(
                ea_hbm.at[pl.ds(base2 + ci * (CH // 2), CH // 2)], ea_v[b],
                sem_ea[b],
            ).wait()
            pltpu.make_async_copy(
                h_hbm.at[src_v[b]], rows_v[r3], sem_g[r3]
            ).wait()

        def wait_scatter(r3):
            pltpu.make_async_copy(
                rows_v[r3], agg_sh.at[dst_v[0]], sem_sc[r3]
            ).wait()

        load_src(0, 0)
        load_src(1, 1)
        load_dst(0, 0)
        load_dst(1, 1)
        wait_src(0, 0)
        start_gather_ea(0, 0, 0)
        pltpu.make_async_copy(
            zeros_hbm, agg_sh.at[pl.ds(s * RPT, RPT)], sem_z
        ).wait()
        plsc.subcore_barrier()

        def slot(ci, j):
            b = j % 2
            r3 = j % 3
            dj = j % 6

            @pl.when(ci + 1 < NCH)
            def _():
                wait_src(ci + 1, 1 - b)
                @pl.when(ci >= 2)
                def _():
                    wait_scatter((r3 + 1) % 3)

                start_gather_ea(ci + 1, 1 - b, (r3 + 1) % 3)

            @pl.when(ci + 2 < NCH)
            def _():
                load_dst(ci + 2, (dj + 2) % 6)

            wait_gather_ea(ci, b, r3)

            @pl.when(ci + 2 < NCH)
            def _():
                load_src(ci + 2, b)

            wait_dst(ci, dj)

            @plsc.parallel_loop(0, CH // 2, 1, unroll=4)
            def _(wr):
                for hh in range(2):
                    r = 2 * wr + hh
                    for g in range(D // 32):
                        w = ea_v[b][wr, pl.ds(hh * 64 + g * 16, 16)]
                        f_lo = plsc.bitcast(
                            lax.shift_left(w, 16), jnp.float32
                        )
                        f_hi = plsc.bitcast(
                            w & jnp.int32(-65536), jnp.float32
                        )
                        sl_lo = pl.ds(g * 32, 16)
                        sl_hi = pl.ds(g * 32 + 16, 16)
                        rows_v[r3][r, sl_lo] = jnp.maximum(
                            rows_v[r3][r, sl_lo] + f_lo, 0.0
                        )
                        rows_v[r3][r, sl_hi] = jnp.maximum(
                            rows_v[r3][r, sl_hi] + f_hi, 0.0
                        )

            pltpu.async_copy(
                rows_v[r3], agg_sh.at[dst_v[dj]], sem_sc[r3], add=True
            )

        def six(q, carry):
            for j in range(6):
                ci = 6 * q + j

                @pl.when(ci < NCH)
                def _():
                    slot(ci, j)
            return carry

        lax.fori_loop(0, _N6, six, 0)

        for r3 in range(3):
            wait_scatter(r3)

        plsc.subcore_barrier()
        pltpu.sync_copy(
            agg_sh.at[pl.ds(s * RPT, RPT)],
            out_hbm.at[c, pl.ds(s * RPT, RPT)],
        )

    return body


_SC_MSG = _make_sc_msg()


def _node_body(eps1_ref, h_ref, agg_ref, wn_ref, bn_ref, g_ref, b_ref, o_ref):
    h2 = eps1_ref[0] * h_ref[...] + (agg_ref[0, :N] + agg_ref[1, :N])
    t = jnp.dot(h2, wn_ref[...], preferred_element_type=jnp.float32) + bn_ref[...]
    mu = jnp.mean(t, axis=0, keepdims=True)
    var = jnp.mean((t - mu) ** 2, axis=0, keepdims=True)
    xn = (t - mu) * lax.rsqrt(var + 1e-5) * g_ref[...] + b_ref[...]
    o_ref[...] = _leaky(_leaky(xn))


def _node_update(eps1, h, agg, wn, bn, gamma, beta):
    return pl.pallas_call(
        _node_body,
        in_specs=[pl.BlockSpec(memory_space=pltpu.SMEM)]
        + [pl.BlockSpec()] * 6,
        out_specs=pl.BlockSpec(),
        out_shape=jax.ShapeDtypeStruct((N, D), jnp.float32),
    )(eps1, h, agg, wn, bn, gamma, beta)


def _node_cls_body(eps1_ref, h_ref, agg_ref, wn_ref, bn_ref, g_ref, b_ref,
                   wc1_ref, bc1_ref, wc_ref, bc_ref, wf_ref, bf_ref, o_ref):
    h2 = eps1_ref[0] * h_ref[...] + (agg_ref[0, :N] + agg_ref[1, :N])
    t = jnp.dot(h2, wn_ref[...], preferred_element_type=jnp.float32) + bn_ref[...]
    mu = jnp.mean(t, axis=0, keepdims=True)
    var = jnp.mean((t - mu) ** 2, axis=0, keepdims=True)
    xn = (t - mu) * lax.rsqrt(var + 1e-5) * g_ref[...] + b_ref[...]
    g = _leaky(_leaky(xn))
    g = jnp.dot(g, wc1_ref[...], preferred_element_type=jnp.float32)
    g = g + bc1_ref[...]
    for i in range(NCL):
        g = jnp.dot(g, wc_ref[i], preferred_element_type=jnp.float32) + bc_ref[i]
        g = _leaky(g)
    z = jnp.dot(g, wf_ref[...], preferred_element_type=jnp.float32) + bf_ref[...]
    o_ref[...] = jax.nn.sigmoid(z)


def _node_cls(eps1, h, agg, wn, bn, gamma, beta, Wc1, bc1, Wc, bc, Wf, bf):
    return pl.pallas_call(
        _node_cls_body,
        in_specs=[pl.BlockSpec(memory_space=pltpu.SMEM)]
        + [pl.BlockSpec()] * 12,
        out_specs=pl.BlockSpec(),
        out_shape=jax.ShapeDtypeStruct((N, 1), jnp.float32),
    )(eps1, h, agg, wn, bn, gamma, beta, Wc1, bc1, Wc, bc, Wf, bf)


def kernel(x, edge_index, edge_attr, batch, We, be, eps, Wn, bn, gamma, beta,
           Wc1, bc1, Wc, bc, Wf, bf):
    src = edge_index[0]
    dst = edge_index[1]
    zeros = jnp.zeros((RPT, D), jnp.float32)
    pl_idx = jnp.array(_PL, dtype=jnp.int32)
    ph_idx = jnp.array(_PH, dtype=jnp.int32)
    ea2 = edge_attr.reshape(E // 2, 2 * ED)
    z = jnp.zeros((ED, D // 2), jnp.float32)
    eas = []
    for i in range(NCONV):
        wpl = We[i][:, pl_idx]
        wph = We[i][:, ph_idx]
        wl = jnp.block([[wpl, z], [z, wpl]])
        wh = jnp.block([[wph, z], [z, wph]])
        bl = jnp.concatenate([be[i][pl_idx], be[i][pl_idx]]).reshape(1, D)
        bh = jnp.concatenate([be[i][ph_idx], be[i][ph_idx]]).reshape(1, D)
        eas.append(_ea_layer(ea2, wl, wh, bl, bh))
    h = x
    for i in range(NCONV - 1):
        agg = _SC_MSG(h, eas[i], src, dst, zeros)
        h = _node_update(
            (1.0 + eps[i]).reshape(1),
            h, agg, Wn[i],
            bn[i].reshape(1, D),
            gamma[i].reshape(1, D),
            beta[i].reshape(1, D),
        )
    i = NCONV - 1
    agg = _SC_MSG(h, eas[i], src, dst, zeros)
    return _node_cls(
        (1.0 + eps[i]).reshape(1),
        h, agg, Wn[i],
        bn[i].reshape(1, D),
        gamma[i].reshape(1, D),
        beta[i].reshape(1, D),
        Wc1, bc1.reshape(1, SCW), Wc, bc, Wf, bf.reshape(1, 1),
    )
